# baseline (speedup 1.0000x reference)
import functools

import jax
import jax.numpy as jnp
from jax import lax
from jax.experimental import pallas as pl
from jax.experimental.pallas import tpu as pltpu
from jax.experimental.pallas import tpu_sc as plsc

B, L, D = 4096, 200, 64
DP = D // 2
METER_VOCAB = 100000

NC, NS, LANES = 2, 16, 16
NW = NC * NS
GW = 128
BBLK = B // NW
N_CHUNKS = L


def _dot_t(t, w):
    return lax.dot_general(t, w, dimension_numbers=(((0,), (0,)), ((), ())),
                           preferred_element_type=jnp.float32)


def _proj_block_kernel(t_ref, w_ref, o_ref):
    o_ref[...] = _dot_t(t_ref[...], w_ref[...]).astype(jnp.bfloat16)


def _proj_bias_kernel(t_ref, w_ref, b_ref, o_ref):
    o_ref[...] = (_dot_t(t_ref[...], w_ref[...])
                  + b_ref[...]).astype(jnp.bfloat16)


def _project_meter(table_t, w):
    return pl.pallas_call(
        _proj_block_kernel,
        out_shape=jax.ShapeDtypeStruct((METER_VOCAB, D), jnp.bfloat16),
    )(table_t, w)


def _project_small(table_t, w, bias=None):
    out_shape = jax.ShapeDtypeStruct((table_t.shape[1], D), jnp.bfloat16)
    if bias is None:
        return pl.pallas_call(_proj_block_kernel, out_shape=out_shape
                              )(table_t, w)
    return pl.pallas_call(_proj_bias_kernel, out_shape=out_shape
                          )(table_t, w, bias)


def _as_i32_pairs(tbl):
    v = tbl.shape[0]
    return lax.bitcast_convert_type(tbl.reshape(v, DP, 2), jnp.int32)


_mesh = plsc.VectorSubcoreMesh(core_axis_name="c", subcore_axis_name="s")

_scratch = []
for _slot in range(2):
    _scratch += [pltpu.VMEM((GW,), jnp.int32)] * 3
    _scratch += [pltpu.VMEM((GW, DP), jnp.int32)] * 3
    _scratch += [pltpu.VMEM((D, GW), jnp.float32)]
_scratch += [pltpu.SemaphoreType.DMA] * 6


@functools.partial(
    pl.kernel,
    out_type=jax.ShapeDtypeStruct((L, D, B), jnp.float32),
    mesh=_mesh,
    compiler_params=pltpu.CompilerParams(use_tc_tiling_on_sc=False,
                                         needs_layout_passes=False),
    scratch_types=_scratch,
)
def _sc_gather_sum(pm_hbm, plb_hbm, pr_hbm, im_hbm, il_hbm, ir_hbm, out_hbm,
                   im0, il0, ir0, bm0, bl0, br0, o0,
                   im1, il1, ir1, bm1, bl1, br1, o1,
                   semi0, semi1, semg0, semg1, semo0, semo1):
    wid = lax.axis_index("s") * NC + lax.axis_index("c")
    idx_v = ((im0, il0, ir0), (im1, il1, ir1))
    bufs = ((bm0, bl0, br0), (bm1, bl1, br1))
    o_v = (o0, o1)
    semi = (semi0, semi1)
    semg = (semg0, semg1)
    semo = (semo0, semo1)
    idx_hbm = (im_hbm, il_hbm, ir_hbm)
    tables = (pm_hbm, plb_hbm, pr_hbm)
    b0 = wid * BBLK

    iota = lax.iota(jnp.int32, LANES)
    col_vecs = [jnp.full((LANES,), dp, jnp.int32) for dp in range(DP)]

    def load_idx(ci, p, is_async):
        for t in range(3):
            src = idx_hbm[t].at[ci, pl.ds(b0, GW)]
            if is_async:
                pltpu.async_copy(src, idx_v[p][t], semi[p])
            else:
                pltpu.sync_copy(src, idx_v[p][t])

    def wait_idx(p):
        for t in range(3):
            pltpu.make_async_copy(idx_hbm[t].at[0, pl.ds(b0, GW)],
                                  idx_v[p][t], semi[p]).wait()

    def fire_gathers(p):
        for t in range(3):
            pltpu.async_copy(tables[t].at[idx_v[p][t]], bufs[p][t], semg[p])

    def drain_gathers(p):
        for t in range(3):
            pltpu.make_async_copy(tables[t].at[idx_v[p][t]], bufs[p][t],
                                  semg[p]).wait()

    load_idx(0, 0, False)
    fire_gathers(0)
    load_idx(1, 1, True)

    @pl.loop(0, N_CHUNKS, step=2)
    def _pair(ci0):
        for p in range(2):
            q = 1 - p
            ci = ci0 + p
            bm, bl, br = bufs[p]

            @pl.when(ci + 1 < N_CHUNKS)
            def _():
                wait_idx(q)
                fire_gathers(q)

            drain_gathers(p)

            @pl.when(ci + 2 < N_CHUNKS)
            def _():
                load_idx(ci + 2, p, True)

            @pl.when(ci >= 2)
            def _():
                pltpu.make_async_copy(out_hbm.at[0, :, pl.ds(0, GW)],
                                      o_v[p], semo[p]).wait()

            @pl.loop(0, GW, step=LANES)
            def _tok(t0):
                rowv = iota + t0
                for dp in range(DP):
                    gm = plsc.load_gather(bm, [rowv, col_vecs[dp]])
                    gl = plsc.load_gather(bl, [rowv, col_vecs[dp]])
                    gr = plsc.load_gather(br, [rowv, col_vecs[dp]])
                    s = (plsc.bitcast(gm, jnp.bfloat16)
                         + plsc.bitcast(gl, jnp.bfloat16)
                         + plsc.bitcast(gr, jnp.bfloat16))
                    lo, hi = plsc.unpack(s, format=plsc.PackFormat.INTERLEAVED)
                    o_v[p][2 * dp, pl.ds(t0, LANES)] = lo
                    o_v[p][2 * dp + 1, pl.ds(t0, LANES)] = hi

            pltpu.async_copy(o_v[p], out_hbm.at[ci, :, pl.ds(b0, GW)],
                             semo[p])

    pltpu.make_async_copy(out_hbm.at[0, :, pl.ds(0, GW)], o0, semo0).wait()
    pltpu.make_async_copy(out_hbm.at[0, :, pl.ds(0, GW)], o1, semo1).wait()


def kernel(meter, length, remainder, meter_table, leng_table, rem_table, W, b):
    pm = _as_i32_pairs(_project_meter(meter_table.T, W[:D]))
    plb = _as_i32_pairs(_project_small(leng_table.T, W[D:2 * D],
                                       b.reshape(1, D)))
    pr = _as_i32_pairs(_project_small(rem_table.T, W[2 * D:]))
    im = meter.astype(jnp.int32).T
    il = length.astype(jnp.int32).T
    ir = remainder.astype(jnp.int32).T
    out = _sc_gather_sum(pm, plb, pr, im, il, ir)
    return jnp.transpose(out, (2, 0, 1))


# speedup vs baseline: 2.1768x; 2.1768x over previous
import functools

import jax
import jax.numpy as jnp
from jax import lax
from jax.experimental import pallas as pl
from jax.experimental.pallas import tpu as pltpu
from jax.experimental.pallas import tpu_sc as plsc

B, L, D = 4096, 200, 64
METER_VOCAB = 100000

NC, NS, LANES = 2, 16, 16
NW = NC * NS
GW = 128
CHUNK = L
ROWS_PER_W = B // NW
_SPLITS = [(0, GW), (GW, CHUNK - GW)]

MXU_BLK = 5000

_PERM = [(p // 32) * 32 + (p % 2) * 16 + (p % 32) // 2 for p in range(D)]


def _dot_t(t, w):
    return lax.dot_general(t, w, dimension_numbers=(((0,), (0,)), ((), ())),
                           preferred_element_type=jnp.float32)


def _proj_block_kernel(t_ref, w_ref, o_ref):
    o_ref[...] = _dot_t(t_ref[...], w_ref[...]).astype(jnp.bfloat16)


def _proj_bias_kernel(t_ref, w_ref, b_ref, o_ref):
    o_ref[...] = (_dot_t(t_ref[...], w_ref[...])
                  + b_ref[...]).astype(jnp.bfloat16)


def _project_meter(table_t, w):
    return pl.pallas_call(
        _proj_block_kernel,
        out_shape=jax.ShapeDtypeStruct((METER_VOCAB, D), jnp.bfloat16),
    )(table_t, w)


def _project_small(table_t, w, bias=None):
    out_shape = jax.ShapeDtypeStruct((table_t.shape[1], D), jnp.bfloat16)
    if bias is None:
        return pl.pallas_call(_proj_block_kernel, out_shape=out_shape
                              )(table_t, w)
    return pl.pallas_call(_proj_bias_kernel, out_shape=out_shape
                          )(table_t, w, bias)


_mesh = plsc.VectorSubcoreMesh(core_axis_name="c", subcore_axis_name="s")

_scratch = []
for _slot in range(2):
    _scratch += [pltpu.VMEM((CHUNK,), jnp.int32)] * 3
    _scratch += [pltpu.VMEM((CHUNK, D), jnp.bfloat16)] * 3
    _scratch += [pltpu.VMEM((CHUNK, D), jnp.float32)]
_scratch += [pltpu.SemaphoreType.DMA] * 6


@functools.partial(
    pl.kernel,
    out_type=jax.ShapeDtypeStruct((B, L, D), jnp.float32),
    mesh=_mesh,
    compiler_params=pltpu.CompilerParams(use_tc_tiling_on_sc=False,
                                         needs_layout_passes=False),
    scratch_types=_scratch,
)
def _sc_gather_sum(pm_hbm, plb_hbm, pr_hbm, im_hbm, il_hbm, ir_hbm, out_hbm,
                   im0, il0, ir0, bm0, bl0, br0, o0,
                   im1, il1, ir1, bm1, bl1, br1, o1,
                   semi0, semi1, semg0, semg1, semo0, semo1):
    wid = lax.axis_index("s") * NC + lax.axis_index("c")
    idx_v = ((im0, il0, ir0), (im1, il1, ir1))
    bufs = ((bm0, bl0, br0), (bm1, bl1, br1))
    o_v = (o0, o1)
    semi = (semi0, semi1)
    semg = (semg0, semg1)
    semo = (semo0, semo1)
    idx_hbm = (im_hbm, il_hbm, ir_hbm)
    tables = (pm_hbm, plb_hbm, pr_hbm)
    row_base = wid * ROWS_PER_W

    def load_idx(b, p, is_async):
        for t in range(3):
            src = idx_hbm[t].at[b]
            if is_async:
                pltpu.async_copy(src, idx_v[p][t], semi[p])
            else:
                pltpu.sync_copy(src, idx_v[p][t])

    def wait_idx(p):
        for t in range(3):
            pltpu.make_async_copy(idx_hbm[t].at[0], idx_v[p][t],
                                  semi[p]).wait()

    def fire_gathers(p):
        for t in range(3):
            for (off, n) in _SPLITS:
                pltpu.async_copy(
                    tables[t].at[idx_v[p][t].at[pl.ds(off, n)]],
                    bufs[p][t].at[pl.ds(off, n)],
                    semg[p])

    def drain_gathers(p):
        for t in range(3):
            for (off, n) in _SPLITS:
                pltpu.make_async_copy(
                    tables[t].at[idx_v[p][t].at[pl.ds(off, n)]],
                    bufs[p][t].at[pl.ds(off, n)],
                    semg[p]).wait()

    load_idx(row_base, 0, False)
    fire_gathers(0)
    load_idx(row_base + 1, 1, True)

    @pl.loop(0, ROWS_PER_W, step=2)
    def _pair(ci0):
        for p in range(2):
            q = 1 - p
            ci = ci0 + p
            bm, bl, br = bufs[p]

            @pl.when(ci + 1 < ROWS_PER_W)
            def _():
                wait_idx(q)
                fire_gathers(q)

            drain_gathers(p)

            @pl.when(ci + 2 < ROWS_PER_W)
            def _():
                load_idx(row_base + ci + 2, p, True)

            @pl.when(ci >= 2)
            def _():
                pltpu.make_async_copy(out_hbm.at[0], o_v[p], semo[p]).wait()

            @pl.loop(0, CHUNK, step=2)
            def _row(r):
                for rr in range(2):
                    for j in range(D // 32):
                        sl = (r + rr, pl.ds(j * 32, 32))
                        s = bm[sl] + bl[sl] + br[sl]
                        lo, hi = plsc.unpack(
                            s, format=plsc.PackFormat.INTERLEAVED)
                        o_v[p][r + rr, pl.ds(j * 32, LANES)] = lo
                        o_v[p][r + rr, pl.ds(j * 32 + LANES, LANES)] = hi

            pltpu.async_copy(o_v[p], out_hbm.at[row_base + ci], semo[p])

    pltpu.make_async_copy(out_hbm.at[0], o0, semo0).wait()
    pltpu.make_async_copy(out_hbm.at[0], o1, semo1).wait()


def kernel(meter, length, remainder, meter_table, leng_table, rem_table, W, b):
    perm = jnp.asarray(_PERM, dtype=jnp.int32)
    Wp = W[:, perm]
    bp = b[perm]
    pm = _project_meter(meter_table.T, Wp[:D])
    plb = _project_small(leng_table.T, Wp[D:2 * D], bp.reshape(1, D))
    pr = _project_small(rem_table.T, Wp[2 * D:])
    im = meter.astype(jnp.int32)
    il = length.astype(jnp.int32)
    ir = remainder.astype(jnp.int32)
    return _sc_gather_sum(pm, plb, pr, im, il, ir)


# speedup vs baseline: 2.1904x; 1.0062x over previous
import functools

import jax
import jax.numpy as jnp
from jax import lax
from jax.experimental import pallas as pl
from jax.experimental.pallas import tpu as pltpu
from jax.experimental.pallas import tpu_sc as plsc

B, L, D = 4096, 200, 64
METER_VOCAB = 100000

NC, NS, LANES = 2, 16, 16
NW = NC * NS
GW = 128
CHUNK = L
ROWS_PER_W = B // NW
_SPLITS = [(0, GW), (GW, CHUNK - GW)]

MXU_BLK = 5000

_PERM = [(p // 32) * 32 + (p % 2) * 16 + (p % 32) // 2 for p in range(D)]


def _dot_t(t, w):
    return lax.dot_general(t, w, dimension_numbers=(((0,), (0,)), ((), ())),
                           preferred_element_type=jnp.float32)


def _proj_block_kernel(t_ref, w_ref, o_ref):
    o_ref[...] = _dot_t(t_ref[...], w_ref[...]).astype(jnp.bfloat16)


def _proj_bias_kernel(t_ref, w_ref, b_ref, o_ref):
    o_ref[...] = (_dot_t(t_ref[...], w_ref[...])
                  + b_ref[...]).astype(jnp.bfloat16)


def _project_meter(table_t, w):
    return pl.pallas_call(
        _proj_block_kernel,
        out_shape=jax.ShapeDtypeStruct((METER_VOCAB, D), jnp.bfloat16),
    )(table_t, w)


def _project_small(table_t, w, bias=None):
    out_shape = jax.ShapeDtypeStruct((table_t.shape[1], D), jnp.bfloat16)
    if bias is None:
        return pl.pallas_call(_proj_block_kernel, out_shape=out_shape
                              )(table_t, w)
    return pl.pallas_call(_proj_bias_kernel, out_shape=out_shape
                          )(table_t, w, bias)


_mesh = plsc.VectorSubcoreMesh(core_axis_name="c", subcore_axis_name="s")

_scratch = []
for _slot in range(2):
    _scratch += [pltpu.VMEM((CHUNK,), jnp.int32)] * 3
    _scratch += [pltpu.VMEM((CHUNK, D), jnp.bfloat16)] * 3
    _scratch += [pltpu.VMEM((CHUNK, D), jnp.float32)]
_scratch += [pltpu.SemaphoreType.DMA] * 6


@functools.partial(
    pl.kernel,
    out_type=jax.ShapeDtypeStruct((B, L, D), jnp.float32),
    mesh=_mesh,
    compiler_params=pltpu.CompilerParams(use_tc_tiling_on_sc=False,
                                         needs_layout_passes=False),
    scratch_types=_scratch,
)
def _sc_gather_sum(pm_hbm, plb_hbm, pr_hbm, im_hbm, il_hbm, ir_hbm, out_hbm,
                   im0, il0, ir0, bm0, bl0, br0, o0,
                   im1, il1, ir1, bm1, bl1, br1, o1,
                   semi0, semi1, semg0, semg1, semo0, semo1):
    wid = lax.axis_index("s") * NC + lax.axis_index("c")
    idx_v = ((im0, il0, ir0), (im1, il1, ir1))
    bufs = ((bm0, bl0, br0), (bm1, bl1, br1))
    o_v = (o0, o1)
    semi = (semi0, semi1)
    semg = (semg0, semg1)
    semo = (semo0, semo1)
    idx_hbm = (im_hbm, il_hbm, ir_hbm)
    tables = (pm_hbm, plb_hbm, pr_hbm)
    row_base = wid * ROWS_PER_W

    def load_idx(b, p, is_async):
        for t in range(3):
            src = idx_hbm[t].at[b]
            if is_async:
                pltpu.async_copy(src, idx_v[p][t], semi[p])
            else:
                pltpu.sync_copy(src, idx_v[p][t])

    def wait_idx(p):
        for t in range(3):
            pltpu.make_async_copy(idx_hbm[t].at[0], idx_v[p][t],
                                  semi[p]).wait()

    def fire_gathers(p):
        for t in range(3):
            for (off, n) in _SPLITS:
                pltpu.async_copy(
                    tables[t].at[idx_v[p][t].at[pl.ds(off, n)]],
                    bufs[p][t].at[pl.ds(off, n)],
                    semg[p])

    def drain_gathers(p):
        for t in range(3):
            for (off, n) in _SPLITS:
                pltpu.make_async_copy(
                    tables[t].at[idx_v[p][t].at[pl.ds(off, n)]],
                    bufs[p][t].at[pl.ds(off, n)],
                    semg[p]).wait()

    load_idx(row_base, 0, False)
    fire_gathers(0)
    load_idx(row_base + 1, 1, True)

    @pl.loop(0, ROWS_PER_W, step=2)
    def _pair(ci0):
        for p in range(2):
            q = 1 - p
            ci = ci0 + p
            bm, bl, br = bufs[p]

            @pl.when(ci + 1 < ROWS_PER_W)
            def _():
                wait_idx(q)
                fire_gathers(q)

            drain_gathers(p)

            @pl.when(ci + 2 < ROWS_PER_W)
            def _():
                load_idx(row_base + ci + 2, p, True)

            @pl.when(ci >= 2)
            def _():
                pltpu.make_async_copy(out_hbm.at[0], o_v[p], semo[p]).wait()

            @pl.loop(0, CHUNK, step=8)
            def _row(r):
                for rr in range(8):
                    for j in range(D // 32):
                        sl = (r + rr, pl.ds(j * 32, 32))
                        s = bm[sl] + bl[sl] + br[sl]
                        lo, hi = plsc.unpack(
                            s, format=plsc.PackFormat.INTERLEAVED)
                        o_v[p][r + rr, pl.ds(j * 32, LANES)] = lo
                        o_v[p][r + rr, pl.ds(j * 32 + LANES, LANES)] = hi

            pltpu.async_copy(o_v[p], out_hbm.at[row_base + ci], semo[p])

    pltpu.make_async_copy(out_hbm.at[0], o0, semo0).wait()
    pltpu.make_async_copy(out_hbm.at[0], o1, semo1).wait()


def kernel(meter, length, remainder, meter_table, leng_table, rem_table, W, b):
    perm = jnp.asarray(_PERM, dtype=jnp.int32)
    Wp = W[:, perm]
    bp = b[perm]
    pm = _project_meter(meter_table.T, Wp[:D])
    plb = _project_small(leng_table.T, Wp[D:2 * D], bp.reshape(1, D))
    pr = _project_small(rem_table.T, Wp[2 * D:])
    im = meter.astype(jnp.int32)
    il = length.astype(jnp.int32)
    ir = remainder.astype(jnp.int32)
    return _sc_gather_sum(pm, plb, pr, im, il, ir)
